# baseline (speedup 1.0000x reference)
import functools

import jax
import jax.numpy as jnp
from jax.experimental import pallas as pl


def _expert_mlp_kernel(xt_ref, wi_ref, wo_ref, m0_ref, m1_ref, r0_ref, r1_ref,
                       o_ref, *, bt, nt):
    e = pl.program_id(0)
    f = pl.program_id(1)

    @pl.when((e == 0) & (f == 0))
    def _():
        o_ref[...] = jnp.zeros_like(o_ref)

    wib = wi_ref[0].astype(jnp.bfloat16)
    wob = wo_ref[0].astype(jnp.bfloat16)

    def mm1(t):
        cols = pl.ds(t * bt, bt)
        x = xt_ref[:, cols]
        h = jax.lax.dot_general(wib, x, (((1,), (0,)), ((), ())),
                                preferred_element_type=jnp.float32)
        return jnp.maximum(h.astype(jnp.bfloat16), jnp.bfloat16(0.0))

    def mm2_accum(t, h):
        cols = pl.ds(t * bt, bt)
        o = jax.lax.dot_general(wob, h, (((1,), (0,)), ((), ())),
                                preferred_element_type=jnp.float32)
        call = (m0_ref[:, cols] * r0_ref[:, cols]
                + m1_ref[:, cols] * r1_ref[:, cols])
        onehot = jax.lax.broadcasted_iota(jnp.int32, call.shape, 0) == e
        c = jnp.sum(jnp.where(onehot, call, 0.0), axis=0, keepdims=True)
        o_ref[:, cols] += o * c

    h_prev = mm1(0)
    for t in range(1, nt):
        h_cur = mm1(t)
        mm2_accum(t - 1, h_prev)
        h_prev = h_cur
    mm2_accum(nt - 1, h_prev)


def kernel(hidden_states, selected_experts, routing_weights, wi, wo):
    T, D = hidden_states.shape
    E, F, _ = wi.shape

    xt = hidden_states.astype(jnp.bfloat16).T
    maskf = selected_experts.astype(jnp.float32)
    m0 = maskf[:, 0, :].T
    m1 = maskf[:, 1, :].T
    r0 = routing_weights[:, 0:1].T
    r1 = routing_weights[:, 1:2].T

    BT = 512
    BF = 1536
    NT = T // BT
    NF = F // BF

    body = functools.partial(_expert_mlp_kernel, bt=BT, nt=NT)

    out_t = pl.pallas_call(
        body,
        grid=(E, NF),
        in_specs=[
            pl.BlockSpec((D, T), lambda e, f: (0, 0)),
            pl.BlockSpec((1, BF, D), lambda e, f: (e, f, 0)),
            pl.BlockSpec((1, D, BF), lambda e, f: (e, 0, f)),
            pl.BlockSpec((E, T), lambda e, f: (0, 0)),
            pl.BlockSpec((E, T), lambda e, f: (0, 0)),
            pl.BlockSpec((1, T), lambda e, f: (0, 0)),
            pl.BlockSpec((1, T), lambda e, f: (0, 0)),
        ],
        out_specs=pl.BlockSpec((D, T), lambda e, f: (0, 0)),
        out_shape=jax.ShapeDtypeStruct((D, T), jnp.float32),
    )(xt, wi, wo, m0, m1, r0, r1)
    return out_t.T


# speedup vs baseline: 1.0694x; 1.0694x over previous
import functools

import jax
import jax.numpy as jnp
from jax.experimental import pallas as pl
from jax.experimental.pallas import tpu as pltpu


def _expert_mlp_kernel(xb_ref, wi_ref, wo_ref, m0_ref, m1_ref, r0_ref, r1_ref,
                       o_ref, *, bt, nt):
    e = pl.program_id(0)
    f = pl.program_id(1)

    @pl.when((e == 0) & (f == 0))
    def _():
        o_ref[...] = jnp.zeros_like(o_ref)

    wib = wi_ref[0].astype(jnp.bfloat16)
    wob = wo_ref[0].astype(jnp.bfloat16)

    def mm1(t):
        rows = pl.ds(t * bt, bt)
        x = xb_ref[rows, :]
        h = jax.lax.dot_general(x, wib, (((1,), (1,)), ((), ())),
                                preferred_element_type=jnp.float32)
        return jnp.maximum(h.astype(jnp.bfloat16), jnp.bfloat16(0.0))

    def mm2_accum(t, h):
        rows = pl.ds(t * bt, bt)
        o = jax.lax.dot_general(h, wob, (((1,), (1,)), ((), ())),
                                preferred_element_type=jnp.float32)
        call = (m0_ref[rows, :] * r0_ref[rows, :]
                + m1_ref[rows, :] * r1_ref[rows, :])
        onehot = jax.lax.broadcasted_iota(jnp.int32, call.shape, 1) == e
        c = jnp.sum(jnp.where(onehot, call, 0.0), axis=1, keepdims=True)
        o_ref[rows, :] += o * c

    h_prev = mm1(0)
    for t in range(1, nt):
        h_cur = mm1(t)
        mm2_accum(t - 1, h_prev)
        h_prev = h_cur
    mm2_accum(nt - 1, h_prev)


def kernel(hidden_states, selected_experts, routing_weights, wi, wo):
    T, D = hidden_states.shape
    E, F, _ = wi.shape

    xb = hidden_states.astype(jnp.bfloat16)
    maskf = selected_experts.astype(jnp.float32)
    m0 = maskf[:, 0, :]
    m1 = maskf[:, 1, :]
    r0 = routing_weights[:, 0:1]
    r1 = routing_weights[:, 1:2]

    BT = 1024
    BF = 1536
    NT = T // BT
    NF = F // BF

    body = functools.partial(_expert_mlp_kernel, bt=BT, nt=NT)

    out = pl.pallas_call(
        body,
        grid=(E, NF),
        in_specs=[
            pl.BlockSpec((T, D), lambda e, f: (0, 0)),
            pl.BlockSpec((1, BF, D), lambda e, f: (e, f, 0)),
            pl.BlockSpec((1, D, BF), lambda e, f: (e, 0, f)),
            pl.BlockSpec((T, E), lambda e, f: (0, 0)),
            pl.BlockSpec((T, E), lambda e, f: (0, 0)),
            pl.BlockSpec((T, 1), lambda e, f: (0, 0)),
            pl.BlockSpec((T, 1), lambda e, f: (0, 0)),
        ],
        out_specs=pl.BlockSpec((T, D), lambda e, f: (0, 0)),
        out_shape=jax.ShapeDtypeStruct((T, D), jnp.float32),
        compiler_params=pltpu.CompilerParams(
            dimension_semantics=("arbitrary", "arbitrary"),
            vmem_limit_bytes=64 * 1024 * 1024,
        ),
    )(xb, wi, wo, m0, m1, r0, r1)
    return out
